# Initial kernel scaffold; baseline (speedup 1.0000x reference)
import jax
import jax.numpy as jnp
from jax.experimental import pallas as pl


def kernel(move_index, table):
    raise NotImplementedError("write your pallas kernel here")


# speedup vs baseline: 3.0382x; 3.0382x over previous
import jax
import jax.numpy as jnp
from jax.experimental import pallas as pl
from jax.experimental.pallas import tpu as pltpu
from jax.experimental.pallas import tpu_sc as plsc

_MOVE_VOCAB_SIZE = 4672
_EMBED_DIM = 128
_BATCH = 4096
_HIST_LEN = 50
_NUM_INDICES = _BATCH * _HIST_LEN
_WINDOW = 128


def _sc_gather(table, flat_idx):
    mesh = plsc.VectorSubcoreMesh(core_axis_name="core", subcore_axis_name="subcore")

    @pl.kernel(
        out_type=jax.ShapeDtypeStruct((_NUM_INDICES, _EMBED_DIM), table.dtype),
        mesh=mesh,
    )
    def kern(tab_hbm, idx_hbm, out_hbm):
        def body(i_vmem, o_vmem):
            pltpu.sync_copy(tab_hbm.at[i_vmem.at[0]], o_vmem)

        pltpu.emit_pipeline(
            body,
            grid=(_NUM_INDICES // _WINDOW,),
            in_specs=[pl.BlockSpec((1, _WINDOW), index_map=lambda i: (0, i))],
            out_specs=[pl.BlockSpec((_WINDOW, _EMBED_DIM), index_map=lambda i: (i, 0))],
            core_axis_name=("core", "subcore"),
            dimension_semantics=(pltpu.PARALLEL,),
        )(idx_hbm, out_hbm)

    return kern(table, flat_idx)


@jax.jit
def kernel(move_index, table):
    flat = move_index.reshape(1, _NUM_INDICES)
    out = _sc_gather(table, flat)
    return jax.lax.stop_gradient(out.reshape(_BATCH, _HIST_LEN, _EMBED_DIM))


# speedup vs baseline: 3.1736x; 1.0446x over previous
import jax
import jax.numpy as jnp
from jax.experimental import pallas as pl
from jax.experimental.pallas import tpu as pltpu
from jax.experimental.pallas import tpu_sc as plsc

_MOVE_VOCAB_SIZE = 4672
_EMBED_DIM = 128
_BATCH = 4096
_HIST_LEN = 50
_NUM_INDICES = _BATCH * _HIST_LEN
_WINDOW = 256


def _sc_gather(table, flat_idx):
    mesh = plsc.VectorSubcoreMesh(core_axis_name="core", subcore_axis_name="subcore")

    @pl.kernel(
        out_type=jax.ShapeDtypeStruct((_NUM_INDICES, _EMBED_DIM), table.dtype),
        mesh=mesh,
    )
    def kern(tab_hbm, idx_hbm, out_hbm):
        def body(i_vmem, o_vmem):
            pltpu.sync_copy(tab_hbm.at[i_vmem.at[0]], o_vmem)

        pltpu.emit_pipeline(
            body,
            grid=(_NUM_INDICES // _WINDOW,),
            in_specs=[pl.BlockSpec((1, _WINDOW), index_map=lambda i: (0, i))],
            out_specs=[pl.BlockSpec((_WINDOW, _EMBED_DIM), index_map=lambda i: (i, 0))],
            core_axis_name=("core", "subcore"),
            dimension_semantics=(pltpu.PARALLEL,),
        )(idx_hbm, out_hbm)

    return kern(table, flat_idx)


@jax.jit
def kernel(move_index, table):
    flat = move_index.reshape(1, _NUM_INDICES)
    out = _sc_gather(table, flat)
    return jax.lax.stop_gradient(out.reshape(_BATCH, _HIST_LEN, _EMBED_DIM))


# speedup vs baseline: 3.6237x; 1.1418x over previous
import jax
import jax.numpy as jnp
from jax import lax
from jax.experimental import pallas as pl
from jax.experimental.pallas import tpu as pltpu
from jax.experimental.pallas import tpu_sc as plsc

_MOVE_VOCAB_SIZE = 4672
_EMBED_DIM = 128
_BATCH = 4096
_HIST_LEN = 50
_NUM_INDICES = _BATCH * _HIST_LEN
_WINDOW = 256

_NUM_SUBCORES = 16
_STAGE_MAIN = 288
_STAGE_REM_BASE = _STAGE_MAIN * _NUM_SUBCORES
_STAGE_REM = _MOVE_VOCAB_SIZE - _STAGE_REM_BASE


def _sc_gather(table, flat_idx):
    mesh = plsc.VectorSubcoreMesh(core_axis_name="core", subcore_axis_name="subcore")

    @pl.kernel(
        out_type=jax.ShapeDtypeStruct((_NUM_INDICES, _EMBED_DIM), table.dtype),
        mesh=mesh,
        scratch_types=[
            pltpu.VMEM_SHARED((_MOVE_VOCAB_SIZE, _EMBED_DIM), jnp.float32),
        ],
    )
    def kern(tab_hbm, idx_hbm, out_hbm, tab_sp):
        sid = lax.axis_index("subcore")
        base = sid * _STAGE_MAIN
        pltpu.sync_copy(
            tab_hbm.at[pl.ds(base, _STAGE_MAIN)],
            tab_sp.at[pl.ds(base, _STAGE_MAIN)],
        )

        @pl.when(sid < _STAGE_REM // 8)
        def _():
            rbase = _STAGE_REM_BASE + sid * 8
            pltpu.sync_copy(
                tab_hbm.at[pl.ds(rbase, 8)],
                tab_sp.at[pl.ds(rbase, 8)],
            )

        plsc.subcore_barrier()

        def body(i_vmem, o_vmem):
            pltpu.sync_copy(tab_sp.at[i_vmem.at[0]], o_vmem)

        pltpu.emit_pipeline(
            body,
            grid=(_NUM_INDICES // _WINDOW,),
            in_specs=[pl.BlockSpec((1, _WINDOW), index_map=lambda i: (0, i))],
            out_specs=[pl.BlockSpec((_WINDOW, _EMBED_DIM), index_map=lambda i: (i, 0))],
            core_axis_name=("core", "subcore"),
            dimension_semantics=(pltpu.PARALLEL,),
        )(idx_hbm, out_hbm)

    return kern(table, flat_idx)


@jax.jit
def kernel(move_index, table):
    flat = move_index.reshape(1, _NUM_INDICES)
    out = _sc_gather(table, flat)
    return jax.lax.stop_gradient(out.reshape(_BATCH, _HIST_LEN, _EMBED_DIM))


# speedup vs baseline: 6.8012x; 1.8769x over previous
import jax
import jax.numpy as jnp
from jax import lax
from jax.experimental import pallas as pl
from jax.experimental.pallas import tpu as pltpu
from jax.experimental.pallas import tpu_sc as plsc

_MOVE_VOCAB_SIZE = 4672
_EMBED_DIM = 128
_BATCH = 4096
_HIST_LEN = 50
_B_BLK = 4

_NUM_SUBCORES = 16
_STAGE_MAIN = 288
_STAGE_REM_BASE = _STAGE_MAIN * _NUM_SUBCORES
_STAGE_REM = _MOVE_VOCAB_SIZE - _STAGE_REM_BASE


def _sc_gather(table, idx3):
    mesh = plsc.VectorSubcoreMesh(core_axis_name="core", subcore_axis_name="subcore")

    @pl.kernel(
        out_type=jax.ShapeDtypeStruct((_BATCH, _HIST_LEN, _EMBED_DIM), table.dtype),
        mesh=mesh,
        scratch_types=[
            pltpu.VMEM_SHARED((_MOVE_VOCAB_SIZE, _EMBED_DIM), jnp.float32),
        ],
    )
    def kern(tab_hbm, idx_hbm, out_hbm, tab_sp):
        sid = lax.axis_index("subcore")
        base = sid * _STAGE_MAIN
        pltpu.sync_copy(
            tab_hbm.at[pl.ds(base, _STAGE_MAIN)],
            tab_sp.at[pl.ds(base, _STAGE_MAIN)],
        )

        @pl.when(sid < _STAGE_REM // 8)
        def _():
            rbase = _STAGE_REM_BASE + sid * 8
            pltpu.sync_copy(
                tab_hbm.at[pl.ds(rbase, 8)],
                tab_sp.at[pl.ds(rbase, 8)],
            )

        plsc.subcore_barrier()

        def body(i_vmem, o_vmem):
            for b in range(_B_BLK):
                pltpu.sync_copy(tab_sp.at[i_vmem.at[b, 0]], o_vmem.at[b])

        pltpu.emit_pipeline(
            body,
            grid=(_BATCH // _B_BLK,),
            in_specs=[
                pl.BlockSpec((_B_BLK, 1, _HIST_LEN), index_map=lambda i: (i, 0, 0))
            ],
            out_specs=[
                pl.BlockSpec(
                    (_B_BLK, _HIST_LEN, _EMBED_DIM), index_map=lambda i: (i, 0, 0)
                )
            ],
            core_axis_name=("core", "subcore"),
            dimension_semantics=(pltpu.PARALLEL,),
        )(idx_hbm, out_hbm)

    return kern(table, idx3)


@jax.jit
def kernel(move_index, table):
    idx3 = move_index.reshape(_BATCH, 1, _HIST_LEN)
    return jax.lax.stop_gradient(_sc_gather(table, idx3))
